# Initial kernel scaffold; baseline (speedup 1.0000x reference)
import jax
import jax.numpy as jnp
from jax.experimental import pallas as pl


def kernel(features, proposals, w_fc6, b_fc6, w_fc7, b_fc7, w_cls, b_cls, w_bbox, b_bbox):
    raise NotImplementedError("write your pallas kernel here")


# speedup vs baseline: 9.6798x; 9.6798x over previous
import jax
import jax.numpy as jnp
from jax.experimental import pallas as pl
from jax.experimental.pallas import tpu as pltpu

_B = 8
_B2 = 40
_N = 1000
_NP = 1024
_H = 100
_W = 100
_C = 256
_ROI = 7
_SR = 2
_STRIDE = 8
_NC = 21
_DH = 256
_DIN = _C * _ROI * _ROI
_S = _ROI * _SR
_BS = _B * _S


def _interp_mat(base, binsz, lim):
    rowid = jax.lax.broadcasted_iota(jnp.int32, (_BS, _B), 0)
    colid = jax.lax.broadcasted_iota(jnp.int32, (_BS, _B), 1)
    R = (rowid // _S == colid).astype(jnp.float32)
    base_r = jnp.dot(R, base, preferred_element_type=jnp.float32)
    bin_r = jnp.dot(R, binsz, preferred_element_type=jnp.float32)
    s = jax.lax.broadcasted_iota(jnp.int32, (_BS, 1), 0) % _S
    frac = (s.astype(jnp.float32) + 0.5) / _SR
    cs = jnp.clip(base_r + bin_r * frac, 0.0, lim - 1.0)
    c0 = jnp.floor(cs)
    lc = cs - c0
    c1 = jnp.minimum(c0 + 1.0, lim - 1.0)
    col = jax.lax.broadcasted_iota(jnp.int32, (_BS, lim), 1).astype(jnp.float32)
    m = (col == c0).astype(jnp.float32) * (1.0 - lc)
    m = m + (col == c1).astype(jnp.float32) * lc
    m = m.reshape(_B * _ROI, _SR, lim)
    return (m[:, 0, :] + m[:, 1, :]) * 0.5


def _pool_kernel(featT_ref, props_ref, out_ref):
    props = props_ref[...]
    scale = 1.0 / _STRIDE
    bx1 = props[:, 0:1] * scale
    by1 = props[:, 1:2] * scale
    roi_w = jnp.maximum(props[:, 2:3] * scale - bx1, 1.0)
    roi_h = jnp.maximum(props[:, 3:4] * scale - by1, 1.0)
    Ax = _interp_mat(bx1, roi_w / _ROI, _W)
    Ay = _interp_mat(by1, roi_h / _ROI, _H)

    s1 = jax.lax.dot_general(Ay, featT_ref[...], (((1,), (0,)), ((), ())),
                             preferred_element_type=jnp.float32)
    s1 = s1.reshape(_B * _ROI, _W, _C)
    Ax3 = Ax.reshape(_B, _ROI, _W)

    outs = []
    for n in range(_B):
        axn = Ax3[n]
        for p in range(_ROI):
            rn = s1[n * _ROI + p]
            outs.append(jnp.dot(axn, rn, preferred_element_type=jnp.float32))
    out_ref[...] = jnp.concatenate(outs, axis=0)


def _fc_kernel(flat_ref, props_ref, w6_ref, b6_ref, w7_ref, b7_ref,
               wc_ref, bc_ref, wb_ref, bb_ref,
               selb_ref, sels_ref, lab_ref):
    flat = flat_ref[...]
    props = props_ref[...]
    h = jnp.dot(flat, w6_ref[...], preferred_element_type=jnp.float32) + b6_ref[...]
    h = jnp.maximum(h, 0.0)
    h = jnp.dot(h, w7_ref[...], preferred_element_type=jnp.float32) + b7_ref[...]
    h = jnp.maximum(h, 0.0)
    logits = jnp.dot(h, wc_ref[...], preferred_element_type=jnp.float32) + bc_ref[...]
    deltas = jnp.dot(h, wb_ref[...], preferred_element_type=jnp.float32) + bb_ref[...]

    mx = jnp.max(logits, axis=1, keepdims=True)
    e = jnp.exp(logits - mx)
    probs = e / jnp.sum(e, axis=1, keepdims=True)
    pfg = probs[:, 1:]
    sel_s = jnp.max(pfg, axis=1, keepdims=True)
    idx = jax.lax.broadcasted_iota(jnp.int32, (_B2, _NC - 1), 1).astype(jnp.float32)
    big = jnp.where(pfg == sel_s, idx, 1e9)
    best = jnp.min(big, axis=1, keepdims=True)
    idx21 = jax.lax.broadcasted_iota(jnp.int32, (_B2, _NC), 1).astype(jnp.float32)
    oh21 = (idx21 == best + 1.0).astype(jnp.float32)

    sel = []
    for k in range(4):
        dk = deltas[:, k * _NC:(k + 1) * _NC]
        sel.append(jnp.sum(dk * oh21, axis=1, keepdims=True))

    pw = props[:, 2:3] - props[:, 0:1]
    ph = props[:, 3:4] - props[:, 1:2]
    pcx = props[:, 0:1] + 0.5 * pw
    pcy = props[:, 1:2] + 0.5 * ph
    cx = sel[0] * 0.1 * pw + pcx
    cy = sel[1] * 0.1 * ph + pcy
    w = jnp.exp(sel[2] * 0.2) * pw
    hh = jnp.exp(sel[3] * 0.2) * ph
    img_w = float(_W * _STRIDE)
    img_h = float(_H * _STRIDE)
    x1o = jnp.clip(cx - 0.5 * w, 0.0, img_w)
    y1o = jnp.clip(cy - 0.5 * hh, 0.0, img_h)
    x2o = jnp.clip(cx + 0.5 * w, 0.0, img_w)
    y2o = jnp.clip(cy + 0.5 * hh, 0.0, img_h)
    selb_ref[...] = jnp.concatenate([x1o, y1o, x2o, y2o], axis=1)
    sels_ref[...] = sel_s
    lab_ref[...] = best + 1.0


def _nms_kernel(order_ref, nb_ref, v_ref, sb_ref, ss_ref, lab_ref,
                keep_ref, sbs_ref, sss_ref, labs_ref,
                bs_s, area_s, act_s):
    order = order_ref[...]
    colid = jax.lax.broadcasted_iota(jnp.int32, (_NP, _NP), 1)
    P = (colid == order).astype(jnp.float32)
    bs = jnp.dot(P, nb_ref[...], preferred_element_type=jnp.float32)
    vs = jnp.dot(P, v_ref[...], preferred_element_type=jnp.float32)
    sbs_ref[...] = jnp.dot(P, sb_ref[...], preferred_element_type=jnp.float32)
    sss_ref[...] = jnp.dot(P, ss_ref[...], preferred_element_type=jnp.float32)
    labs_ref[...] = jnp.dot(P, lab_ref[...], preferred_element_type=jnp.float32)

    x1 = bs[:, 0:1]
    y1 = bs[:, 1:2]
    x2 = bs[:, 2:3]
    y2 = bs[:, 3:4]
    areas = (x2 - x1) * (y2 - y1)
    bs_s[...] = bs
    area_s[...] = areas
    act_s[...] = vs
    pos = jax.lax.broadcasted_iota(jnp.int32, (_NP, 1), 0)

    def body(i, carry):
        br = bs_s[pl.ds(i, 1), :]
        ai = act_s[pl.ds(i, 1), :]
        ar_i = area_s[pl.ds(i, 1), :]
        keep_ref[pl.ds(i, 1), :] = ai
        xx1 = jnp.maximum(br[:, 0:1], x1)
        yy1 = jnp.maximum(br[:, 1:2], y1)
        xx2 = jnp.minimum(br[:, 2:3], x2)
        yy2 = jnp.minimum(br[:, 3:4], y2)
        iw = jnp.maximum(0.0, xx2 - xx1)
        ih = jnp.maximum(0.0, yy2 - yy1)
        inter = iw * ih
        iou = inter / (ar_i + areas - inter + 1e-9)
        supp = ai * (iou > 0.5).astype(jnp.float32) * (pos > i).astype(jnp.float32)
        act_s[...] = act_s[...] * (1.0 - supp)
        return carry

    jax.lax.fori_loop(0, _NP, body, 0)


def kernel(features, proposals, w_fc6, b_fc6, w_fc7, b_fc7, w_cls, b_cls,
           w_bbox, b_bbox):
    featT = jnp.transpose(features[0], (1, 2, 0)).reshape(_H, _W * _C)
    w6r = w_fc6.reshape(_C, _ROI, _ROI, _DH).transpose(1, 2, 0, 3).reshape(_DIN, _DH)
    wbr = w_bbox.reshape(_DH, _NC, 4).transpose(0, 2, 1).reshape(_DH, _NC * 4)
    bbr = b_bbox.reshape(_NC, 4).T.reshape(_NC * 4)

    pooled = pl.pallas_call(
        _pool_kernel,
        grid=(_N // _B,),
        in_specs=[
            pl.BlockSpec((_H, _W * _C), lambda i: (0, 0)),
            pl.BlockSpec((_B, 4), lambda i: (i, 0)),
        ],
        out_specs=pl.BlockSpec((_B * _ROI * _ROI, _C), lambda i: (i, 0)),
        out_shape=jax.ShapeDtypeStruct((_N * _ROI * _ROI, _C), jnp.float32),
    )(featT, proposals)
    flat = pooled.reshape(_N, _DIN)

    selb, sels, lab = pl.pallas_call(
        _fc_kernel,
        grid=(_N // _B2,),
        in_specs=[
            pl.BlockSpec((_B2, _DIN), lambda i: (i, 0)),
            pl.BlockSpec((_B2, 4), lambda i: (i, 0)),
            pl.BlockSpec((_DIN, _DH), lambda i: (0, 0)),
            pl.BlockSpec((1, _DH), lambda i: (0, 0)),
            pl.BlockSpec((_DH, _DH), lambda i: (0, 0)),
            pl.BlockSpec((1, _DH), lambda i: (0, 0)),
            pl.BlockSpec((_DH, _NC), lambda i: (0, 0)),
            pl.BlockSpec((1, _NC), lambda i: (0, 0)),
            pl.BlockSpec((_DH, _NC * 4), lambda i: (0, 0)),
            pl.BlockSpec((1, _NC * 4), lambda i: (0, 0)),
        ],
        out_specs=[
            pl.BlockSpec((_B2, 4), lambda i: (i, 0)),
            pl.BlockSpec((_B2, 1), lambda i: (i, 0)),
            pl.BlockSpec((_B2, 1), lambda i: (i, 0)),
        ],
        out_shape=[
            jax.ShapeDtypeStruct((_N, 4), jnp.float32),
            jax.ShapeDtypeStruct((_N, 1), jnp.float32),
            jax.ShapeDtypeStruct((_N, 1), jnp.float32),
        ],
    )(flat, proposals, w6r, b_fc6[None, :], w_fc7, b_fc7[None, :],
      w_cls, b_cls[None, :], wbr, bbr[None, :])

    sel_b = selb
    sel_s = sels[:, 0]
    labels = lab[:, 0]

    valid = (sel_b[:, 2] > sel_b[:, 0]) & (sel_b[:, 3] > sel_b[:, 1]) & (sel_s > 0.1)
    off = labels[:, None] * (jnp.max(sel_b) + 1.0)
    nms_boxes = sel_b + off
    scores_m = jnp.where(valid, sel_s, -jnp.inf)
    order = jnp.argsort(scores_m)[::-1].astype(jnp.int32)

    pad = _NP - _N
    order_p = jnp.concatenate([order, jnp.arange(_N, _NP, dtype=jnp.int32)])[:, None]
    nb_p = jnp.pad(nms_boxes, ((0, pad), (0, 0)))
    v_p = jnp.pad(valid.astype(jnp.float32)[:, None], ((0, pad), (0, 0)))
    sb_p = jnp.pad(sel_b, ((0, pad), (0, 0)))
    ss_p = jnp.pad(sel_s[:, None], ((0, pad), (0, 0)))
    lab_p = jnp.pad(labels[:, None], ((0, pad), (0, 0)))

    keepf, sbs, sss, labs = pl.pallas_call(
        _nms_kernel,
        out_shape=[
            jax.ShapeDtypeStruct((_NP, 1), jnp.float32),
            jax.ShapeDtypeStruct((_NP, 4), jnp.float32),
            jax.ShapeDtypeStruct((_NP, 1), jnp.float32),
            jax.ShapeDtypeStruct((_NP, 1), jnp.float32),
        ],
        scratch_shapes=[
            pltpu.VMEM((_NP, 4), jnp.float32),
            pltpu.VMEM((_NP, 1), jnp.float32),
            pltpu.VMEM((_NP, 1), jnp.float32),
        ],
    )(order_p, nb_p, v_p, sb_p, ss_p, lab_p)

    keep = keepf[:_N, 0] > 0.5
    rank = jnp.cumsum(keep.astype(jnp.int32)) - 1
    row = jnp.where(keep & (rank < 100), rank, 100)
    data = jnp.concatenate([sbs[:_N], labs[:_N] - 1.0, sss[:_N]], axis=1)
    out = jnp.zeros((101, 6), dtype=jnp.float32)
    out = out.at[row].set(data)
    return out[:100]
